# baseline (device time: 135716 ns/iter reference)
import jax
import jax.numpy as jnp
from jax import lax
from jax.experimental import pallas as pl
from jax.experimental.pallas import tpu as pltpu

N_DEV = 4
M_PER = 1024
K_PER = 1024
BN = 1024

DIST = (None, 1, 3, 2)
SEND_ORDER = (1, 3, 2)


def kernel(x, w_mat):
    k_full, k_per = x.shape
    _, n_full = w_mat.shape
    n_blocks = n_full // BN

    def w_index(p, n):
        my = lax.axis_index("i")
        off = jnp.where(p == 1, 3, jnp.where(p == 2, 1, jnp.where(p == 3, 2, 0)))
        return ((my + off) % N_DEV, n)

    def body(x_hbm, w_ref, out_ref, xg_ref, xsend_ref, stage_ref,
             send_sems, recv_sems, stage_sems):
        p = pl.program_id(0)
        n = pl.program_id(1)
        my = lax.axis_index("i")

        @pl.when((p == 0) & (n == 0))
        def _start():
            barrier_sem = pltpu.get_barrier_semaphore()
            for d in range(1, N_DEV):
                pl.semaphore_signal(
                    barrier_sem, inc=1,
                    device_id=((my + d) % N_DEV,),
                    device_id_type=pl.DeviceIdType.MESH,
                )
            pl.semaphore_wait(barrier_sem, N_DEV - 1)

            pltpu.make_async_copy(
                x_hbm.at[pl.ds(my * M_PER, M_PER), :],
                stage_ref.at[0], stage_sems.at[0],
            ).start()
            tgt = (my + SEND_ORDER[0]) % N_DEV
            pltpu.make_async_copy(
                x_hbm.at[pl.ds(tgt * M_PER, M_PER), :],
                stage_ref.at[1], stage_sems.at[1],
            ).start()
            pltpu.make_async_copy(
                x_hbm.at[pl.ds(my * M_PER, M_PER), :],
                stage_ref.at[0], stage_sems.at[0],
            ).wait()

        for step, d in enumerate(SEND_ORDER):
            @pl.when((p == 0) & (n == step + 1))
            def _send(step=step, d=d):
                tgt = (my + d) % N_DEV
                pltpu.make_async_copy(
                    x_hbm.at[pl.ds(tgt * M_PER, M_PER), :],
                    stage_ref.at[1], stage_sems.at[1],
                ).wait()
                xsend_ref[d - 1] = stage_ref[1].astype(jnp.bfloat16)
                pltpu.make_async_remote_copy(
                    src_ref=xsend_ref.at[d - 1],
                    dst_ref=xg_ref.at[d - 1],
                    send_sem=send_sems.at[d - 1],
                    recv_sem=recv_sems.at[d - 1],
                    device_id=(tgt,),
                    device_id_type=pl.DeviceIdType.MESH,
                ).start()
                if step + 1 < len(SEND_ORDER):
                    nxt = (my + SEND_ORDER[step + 1]) % N_DEV
                    pltpu.make_async_copy(
                        x_hbm.at[pl.ds(nxt * M_PER, M_PER), :],
                        stage_ref.at[1], stage_sems.at[1],
                    ).start()

        for pp in (1, 2, 3):
            d = DIST[pp]

            @pl.when((p == pp) & (n == 0))
            def _wait(d=d):
                pltpu.make_async_remote_copy(
                    src_ref=xsend_ref.at[d - 1],
                    dst_ref=xg_ref.at[d - 1],
                    send_sem=send_sems.at[d - 1],
                    recv_sem=recv_sems.at[d - 1],
                    device_id=(my,),
                    device_id_type=pl.DeviceIdType.MESH,
                ).wait_recv()

        nd = pl.ds(n * BN, BN)

        @pl.when(p == 0)
        def _compute0():
            out_ref[:, nd] = lax.dot_general(
                stage_ref[0], w_ref[:, :],
                (((1,), (0,)), ((), ())),
                precision=lax.Precision.DEFAULT,
                preferred_element_type=jnp.float32,
            )

        for pp in (1, 2, 3):
            d = DIST[pp]

            @pl.when(p == pp)
            def _compute(d=d, pp=pp):
                part = lax.dot_general(
                    xg_ref[d - 1], w_ref[:, :],
                    (((1,), (0,)), ((), ())),
                    precision=lax.Precision.DEFAULT,
                    preferred_element_type=jnp.float32,
                )
                if pp < N_DEV - 1:
                    out_ref[:, nd] = out_ref[:, nd] + part
                else:
                    acc = out_ref[:, nd] + part
                    out_ref[:, nd] = acc * jax.nn.sigmoid(acc)

        @pl.when((p == N_DEV - 1) & (n == n_blocks - 1))
        def _drain():
            for d in range(1, N_DEV):
                pltpu.make_async_remote_copy(
                    src_ref=xsend_ref.at[d - 1],
                    dst_ref=xg_ref.at[d - 1],
                    send_sem=send_sems.at[d - 1],
                    recv_sem=recv_sems.at[d - 1],
                    device_id=(my,),
                    device_id_type=pl.DeviceIdType.MESH,
                ).wait_send()

    return pl.pallas_call(
        body,
        grid=(N_DEV, n_blocks),
        out_shape=jax.ShapeDtypeStruct((M_PER, n_full), jnp.float32),
        in_specs=[
            pl.BlockSpec(memory_space=pltpu.MemorySpace.HBM),
            pl.BlockSpec((K_PER, BN), w_index),
        ],
        out_specs=pl.BlockSpec((M_PER, n_full), lambda p, n: (0, 0)),
        scratch_shapes=[
            pltpu.VMEM((N_DEV - 1, M_PER, K_PER), jnp.bfloat16),
            pltpu.VMEM((N_DEV - 1, M_PER, K_PER), jnp.bfloat16),
            pltpu.VMEM((2, M_PER, K_PER), jnp.float32),
            pltpu.SemaphoreType.DMA((N_DEV - 1,)),
            pltpu.SemaphoreType.DMA((N_DEV - 1,)),
            pltpu.SemaphoreType.DMA((2,)),
        ],
        compiler_params=pltpu.CompilerParams(
            collective_id=0,
            dimension_semantics=("arbitrary", "arbitrary"),
            vmem_limit_bytes=63 * 1024 * 1024 + 512 * 1024,
        ),
    )(x, w_mat)


# device time: 130059 ns/iter; 1.0435x vs baseline; 1.0435x over previous
import jax
import jax.numpy as jnp
from jax import lax
from jax.experimental import pallas as pl
from jax.experimental.pallas import tpu as pltpu

N_DEV = 4
M_PER = 1024
K_PER = 1024
BN = 1024

DIST = (None, 1, 3, 2)
SEND_ORDER = (1, 3, 2)


def kernel(x, w_mat):
    k_full, k_per = x.shape
    _, n_full = w_mat.shape
    n_blocks = n_full // BN

    def w_index(p, n):
        my = lax.axis_index("i")
        off = jnp.where(p == 1, 3, jnp.where(p == 2, 1, jnp.where(p == 3, 2, 0)))
        return ((my + off) % N_DEV, n)

    def body(x_hbm, w_ref, out_hbm, xg_ref, xsend_ref, stage_ref,
             acc_ref, send_sems, recv_sems, stage_sems, out_sem):
        p = pl.program_id(0)
        n = pl.program_id(1)
        my = lax.axis_index("i")

        @pl.when((p == 0) & (n == 0))
        def _start():
            barrier_sem = pltpu.get_barrier_semaphore()
            for d in range(1, N_DEV):
                pl.semaphore_signal(
                    barrier_sem, inc=1,
                    device_id=((my + d) % N_DEV,),
                    device_id_type=pl.DeviceIdType.MESH,
                )
            pl.semaphore_wait(barrier_sem, N_DEV - 1)

            pltpu.make_async_copy(
                x_hbm.at[pl.ds(my * M_PER, M_PER), :],
                stage_ref.at[0], stage_sems.at[0],
            ).start()
            tgt = (my + SEND_ORDER[0]) % N_DEV
            pltpu.make_async_copy(
                x_hbm.at[pl.ds(tgt * M_PER, M_PER), :],
                stage_ref.at[1], stage_sems.at[1],
            ).start()
            pltpu.make_async_copy(
                x_hbm.at[pl.ds(my * M_PER, M_PER), :],
                stage_ref.at[0], stage_sems.at[0],
            ).wait()

        for step, d in enumerate(SEND_ORDER):
            @pl.when((p == 0) & (n == step + 1))
            def _send(step=step, d=d):
                tgt = (my + d) % N_DEV
                pltpu.make_async_copy(
                    x_hbm.at[pl.ds(tgt * M_PER, M_PER), :],
                    stage_ref.at[1], stage_sems.at[1],
                ).wait()
                xsend_ref[d - 1] = stage_ref[1].astype(jnp.bfloat16)
                pltpu.make_async_remote_copy(
                    src_ref=xsend_ref.at[d - 1],
                    dst_ref=xg_ref.at[d - 1],
                    send_sem=send_sems.at[d - 1],
                    recv_sem=recv_sems.at[d - 1],
                    device_id=(tgt,),
                    device_id_type=pl.DeviceIdType.MESH,
                ).start()
                if step + 1 < len(SEND_ORDER):
                    nxt = (my + SEND_ORDER[step + 1]) % N_DEV
                    pltpu.make_async_copy(
                        x_hbm.at[pl.ds(nxt * M_PER, M_PER), :],
                        stage_ref.at[1], stage_sems.at[1],
                    ).start()

        for pp in (1, 2, 3):
            d = DIST[pp]

            @pl.when((p == pp) & (n == 0))
            def _wait(d=d):
                pltpu.make_async_remote_copy(
                    src_ref=xsend_ref.at[d - 1],
                    dst_ref=xg_ref.at[d - 1],
                    send_sem=send_sems.at[d - 1],
                    recv_sem=recv_sems.at[d - 1],
                    device_id=(my,),
                    device_id_type=pl.DeviceIdType.MESH,
                ).wait_recv()

        nd = pl.ds(n * BN, BN)

        @pl.when(p == 0)
        def _compute0():
            acc_ref[:, nd] = lax.dot_general(
                stage_ref[0], w_ref[:, :],
                (((1,), (0,)), ((), ())),
                precision=lax.Precision.DEFAULT,
                preferred_element_type=jnp.float32,
            )

        for pp in (1, 2, 3):
            d = DIST[pp]

            @pl.when(p == pp)
            def _compute(d=d, pp=pp):
                part = lax.dot_general(
                    xg_ref[d - 1], w_ref[:, :],
                    (((1,), (0,)), ((), ())),
                    precision=lax.Precision.DEFAULT,
                    preferred_element_type=jnp.float32,
                )
                if pp < N_DEV - 1:
                    acc_ref[:, nd] = acc_ref[:, nd] + part
                else:
                    acc = acc_ref[:, nd] + part
                    acc_ref[:, nd] = acc * jax.nn.sigmoid(acc)
                    pltpu.make_async_copy(
                        acc_ref.at[:, nd], out_hbm.at[:, nd], out_sem,
                    ).start()

        @pl.when((p == N_DEV - 1) & (n == n_blocks - 1))
        def _drain():
            for k in range(n_blocks):
                kd = pl.ds(k * BN, BN)
                pltpu.make_async_copy(
                    acc_ref.at[:, kd], out_hbm.at[:, kd], out_sem,
                ).wait()
            for d in range(1, N_DEV):
                pltpu.make_async_remote_copy(
                    src_ref=xsend_ref.at[d - 1],
                    dst_ref=xg_ref.at[d - 1],
                    send_sem=send_sems.at[d - 1],
                    recv_sem=recv_sems.at[d - 1],
                    device_id=(my,),
                    device_id_type=pl.DeviceIdType.MESH,
                ).wait_send()

    return pl.pallas_call(
        body,
        grid=(N_DEV, n_blocks),
        out_shape=jax.ShapeDtypeStruct((M_PER, n_full), jnp.float32),
        in_specs=[
            pl.BlockSpec(memory_space=pltpu.MemorySpace.HBM),
            pl.BlockSpec((K_PER, BN), w_index),
        ],
        out_specs=pl.BlockSpec(memory_space=pltpu.MemorySpace.HBM),
        scratch_shapes=[
            pltpu.VMEM((N_DEV - 1, M_PER, K_PER), jnp.bfloat16),
            pltpu.VMEM((N_DEV - 1, M_PER, K_PER), jnp.bfloat16),
            pltpu.VMEM((2, M_PER, K_PER), jnp.float32),
            pltpu.VMEM((M_PER, 8192), jnp.float32),
            pltpu.SemaphoreType.DMA((N_DEV - 1,)),
            pltpu.SemaphoreType.DMA((N_DEV - 1,)),
            pltpu.SemaphoreType.DMA((2,)),
            pltpu.SemaphoreType.DMA,
        ],
        compiler_params=pltpu.CompilerParams(
            collective_id=0,
            dimension_semantics=("arbitrary", "arbitrary"),
            vmem_limit_bytes=63 * 1024 * 1024 + 512 * 1024,
        ),
    )(x, w_mat)


# device time: 128856 ns/iter; 1.0532x vs baseline; 1.0093x over previous
import jax
import jax.numpy as jnp
from jax import lax
from jax.experimental import pallas as pl
from jax.experimental.pallas import tpu as pltpu

N_DEV = 4
M_PER = 1024
K_PER = 1024
BN = 1024

DIST = (None, 1, 3, 2)
SEND_ORDER = (1, 3, 2)


def kernel(x, w_mat):
    k_full, k_per = x.shape
    _, n_full = w_mat.shape
    n_blocks = n_full // BN

    def w_index(p, n):
        my = lax.axis_index("i")
        off = jnp.where(p == 1, 3, jnp.where(p == 2, 1, jnp.where(p == 3, 2, 0)))
        return ((my + off) % N_DEV, n)

    def body(x_hbm, w_ref, out_hbm, xg_ref, xsend_ref, stage_ref,
             acc_ref, send_sems, recv_sems, stage_sems, out_sem):
        p = pl.program_id(0)
        n = pl.program_id(1)
        my = lax.axis_index("i")

        @pl.when((p == 0) & (n == 0))
        def _start():
            barrier_sem = pltpu.get_barrier_semaphore()
            for d in range(1, N_DEV):
                pl.semaphore_signal(
                    barrier_sem, inc=1,
                    device_id=((my + d) % N_DEV,),
                    device_id_type=pl.DeviceIdType.MESH,
                )
            pl.semaphore_wait(barrier_sem, N_DEV - 1)

            pltpu.make_async_copy(
                x_hbm.at[pl.ds(my * M_PER, M_PER), :],
                stage_ref.at[0], stage_sems.at[0],
            ).start()
            tgt = (my + SEND_ORDER[0]) % N_DEV
            pltpu.make_async_copy(
                x_hbm.at[pl.ds(tgt * M_PER, M_PER), :],
                stage_ref.at[1], stage_sems.at[1],
            ).start()
            pltpu.make_async_copy(
                x_hbm.at[pl.ds(my * M_PER, M_PER), :],
                stage_ref.at[0], stage_sems.at[0],
            ).wait()

        for step, d in enumerate(SEND_ORDER):
            @pl.when((p == 0) & (n == step + 1))
            def _send(step=step, d=d):
                tgt = (my + d) % N_DEV
                pltpu.make_async_copy(
                    x_hbm.at[pl.ds(tgt * M_PER, M_PER), :],
                    stage_ref.at[1], stage_sems.at[1],
                ).wait()
                xsend_ref[d - 1] = stage_ref[1].astype(jnp.bfloat16)
                pltpu.make_async_remote_copy(
                    src_ref=xsend_ref.at[d - 1],
                    dst_ref=xg_ref.at[d - 1],
                    send_sem=send_sems.at[d - 1],
                    recv_sem=recv_sems.at[d - 1],
                    device_id=(tgt,),
                    device_id_type=pl.DeviceIdType.MESH,
                ).start()
                if step + 1 < len(SEND_ORDER):
                    nxt = (my + SEND_ORDER[step + 1]) % N_DEV
                    pltpu.make_async_copy(
                        x_hbm.at[pl.ds(nxt * M_PER, M_PER), :],
                        stage_ref.at[1], stage_sems.at[1],
                    ).start()

        for pp in (1, 2, 3):
            d = DIST[pp]

            @pl.when((p == pp) & (n == 0))
            def _wait(d=d):
                pltpu.make_async_remote_copy(
                    src_ref=xsend_ref.at[d - 1],
                    dst_ref=xg_ref.at[d - 1],
                    send_sem=send_sems.at[d - 1],
                    recv_sem=recv_sems.at[d - 1],
                    device_id=(my,),
                    device_id_type=pl.DeviceIdType.MESH,
                ).wait_recv()

        nd = pl.ds(n * BN, BN)

        @pl.when(p == 0)
        def _compute0():
            acc_ref[:, nd] = lax.dot_general(
                stage_ref[0], w_ref[:, :],
                (((1,), (0,)), ((), ())),
                precision=lax.Precision.DEFAULT,
                preferred_element_type=jnp.float32,
            )

        for pp in (1, 2, 3):
            d = DIST[pp]

            @pl.when(p == pp)
            def _compute(d=d, pp=pp):
                part = lax.dot_general(
                    xg_ref[d - 1], w_ref[:, :],
                    (((1,), (0,)), ((), ())),
                    precision=lax.Precision.DEFAULT,
                    preferred_element_type=jnp.float32,
                )
                if pp < N_DEV - 1:
                    acc_ref[:, nd] = acc_ref[:, nd] + part
                else:
                    acc = acc_ref[:, nd] + part
                    acc_ref[:, nd] = acc * jax.nn.sigmoid(acc)
                    pltpu.make_async_copy(
                        acc_ref.at[:, nd], out_hbm.at[:, nd], out_sem,
                    ).start()

        @pl.when((p == N_DEV - 1) & (n == n_blocks - 1))
        def _drain():
            for k in range(n_blocks):
                kd = pl.ds(k * BN, BN)
                pltpu.make_async_copy(
                    acc_ref.at[:, kd], out_hbm.at[:, kd], out_sem,
                ).wait()
            for d in range(1, N_DEV):
                pltpu.make_async_remote_copy(
                    src_ref=xsend_ref.at[d - 1],
                    dst_ref=xg_ref.at[d - 1],
                    send_sem=send_sems.at[d - 1],
                    recv_sem=recv_sems.at[d - 1],
                    device_id=(my,),
                    device_id_type=pl.DeviceIdType.MESH,
                ).wait_send()

    out = pl.pallas_call(
        body,
        grid=(N_DEV, n_blocks),
        out_shape=jax.ShapeDtypeStruct((M_PER, n_full), jnp.float32),
        in_specs=[
            pl.BlockSpec(memory_space=pltpu.MemorySpace.HBM),
            pl.BlockSpec((K_PER, BN), w_index),
        ],
        out_specs=pl.BlockSpec(memory_space=pltpu.MemorySpace.HBM),
        scratch_shapes=[
            pltpu.VMEM((N_DEV - 1, M_PER, K_PER), jnp.bfloat16),
            pltpu.VMEM((N_DEV - 1, M_PER, K_PER), jnp.bfloat16),
            pltpu.VMEM((2, M_PER, K_PER), jnp.float32),
            pltpu.VMEM((M_PER, 8192), jnp.float32),
            pltpu.SemaphoreType.DMA((N_DEV - 1,)),
            pltpu.SemaphoreType.DMA((N_DEV - 1,)),
            pltpu.SemaphoreType.DMA((2,)),
            pltpu.SemaphoreType.DMA,
        ],
        compiler_params=pltpu.CompilerParams(
            collective_id=0,
            dimension_semantics=("arbitrary", "arbitrary"),
            vmem_limit_bytes=63 * 1024 * 1024 + 512 * 1024,
        ),
    )(x, w_mat)
    return out
